# baseline (speedup 1.0000x reference)
import functools

import jax
import jax.numpy as jnp
from jax import lax
from jax.experimental import pallas as pl
from jax.experimental.pallas import tpu as pltpu
from jax.experimental.pallas import tpu_sc as plsc

N_NODES = 10000
N_REL = 500
ALPHA = 0.2

NC = 2
NS = 16
NW = NC * NS

TW = 144
TA = 136
K = 32
E1 = 160000
E2 = 40000
E1P = 163840
E2P = 40960
C1W = E1P // NW // K
C2W = E2P // NW // K
ACC_ROWS = 10240
ZR = ACC_ROWS // NS
MASK_B = 1024
MPW = MASK_B // NW


def _edge_body(nheads, with_mask,
               src2d_hbm, dst2d_hbm, ta2d_hbm, tb2d_hbm, mid_hbm,
               ptab_hbm, rtab_hbm, stab_hbm, out_hbm,
               acc, srcb, dstb, tab, tbb, imask,
               bufA0, bufA1, bufB0, bufB1, bufD0, bufD1, bufC,
               obuf0, obuf1, gs0, gs1, ss0, ss1):
    c = lax.axis_index("c")
    s = lax.axis_index("s")
    wid = c * NS + s
    zoff = pl.multiple_of(s * ZR, 8)
    z16 = jnp.zeros((16,), jnp.float32)

    sets = ((bufA0, bufB0, bufD0, gs0, obuf0, ss0),
            (bufA1, bufB1, bufD1, gs1, obuf1, ss1))

    def zrow(r, _):
        for ob in (obuf0, obuf1):
            for cc in range(8):
                ob[r, pl.ds(cc * 16, 16)] = z16
            ob[r, pl.ds(120, 16)] = z16
        return 0
    lax.fori_loop(0, K, zrow, 0)

    def zsl(i, _):
        pltpu.sync_copy(obuf0, acc.at[pl.ds(pl.multiple_of(zoff + i * K, 8), K)])
        return 0
    lax.fori_loop(0, ZR // K, zsl, 0)
    plsc.subcore_barrier()

    def gissue(k, b, use_c):
        A, B, D, gs, _, _ = sets[b]
        pltpu.async_copy(ptab_hbm.at[dstb.at[k]], A, gs)
        pltpu.async_copy(rtab_hbm.at[tab.at[k]], B, gs)
        pltpu.async_copy(stab_hbm.at[srcb.at[k]], D, gs)

    def gwait(b):
        A, B, D, gs, _, _ = sets[b]
        pltpu.make_async_copy(ptab_hbm.at[dstb.at[0]], A, gs).wait()
        pltpu.make_async_copy(rtab_hbm.at[tab.at[0]], B, gs).wait()
        pltpu.make_async_copy(stab_hbm.at[srcb.at[0]], D, gs).wait()

    def sissue(k, b):
        _, _, _, _, ob, ss = sets[b]
        pltpu.async_copy(ob, acc.at[srcb.at[k]], ss, add=True)

    def swait(b):
        _, _, _, _, ob, ss = sets[b]
        pltpu.make_async_copy(ob, acc.at[srcb.at[0]], ss).wait()

    cph = (128 // 16) // nheads

    def compute(b, use_c):
        A, B, D, _, ob, _ = sets[b]
        for g in range(K // 16):
            rows = g * 16 + lax.iota(jnp.int32, 16)
            wqs = []
            for h in range(nheads):
                colh = jnp.full((16,), 128 + h, jnp.int32)
                logit = (plsc.load_gather(A, [rows, colh])
                         + plsc.load_gather(B, [rows, colh])
                         + plsc.load_gather(D, [rows,
                                                jnp.full((16,), h, jnp.int32)]))
                if use_c:
                    logit = logit + plsc.load_gather(bufC, [rows, colh])
                w = jnp.exp(jnp.where(logit > 0, -logit, -ALPHA * logit))
                plsc.store_scatter(ob, [rows, colh], w)
                wqs.append(w)

        def edge4(q, _):
            for i in range(4):
                r = q * 4 + i
                wv = ob[r, pl.ds(120, 16)]
                wsc = [wv[8 + h] for h in range(nheads)]
                for cc in range(8):
                    v = A[r, pl.ds(cc * 16, 16)] + B[r, pl.ds(cc * 16, 16)]
                    if use_c:
                        v = v + bufC[r, pl.ds(cc * 16, 16)]
                    ob[r, pl.ds(cc * 16, 16)] = v * wsc[cc // cph]
            return 0
        lax.fori_loop(0, K // 4, edge4, 0)

    def run_phase(row0, nhalves, nch, use_c, tbrow0):
        def half(hf, _a):
            r0 = pl.multiple_of(row0 + hf * nch, 8)
            pltpu.sync_copy(src2d_hbm.at[pl.ds(r0, nch)], srcb.at[pl.ds(0, nch)])
            pltpu.sync_copy(dst2d_hbm.at[pl.ds(r0, nch)], dstb.at[pl.ds(0, nch)])
            pltpu.sync_copy(ta2d_hbm.at[pl.ds(r0, nch)], tab.at[pl.ds(0, nch)])
            if use_c:
                pltpu.sync_copy(
                    tb2d_hbm.at[pl.ds(pl.multiple_of(tbrow0 + hf * nch, 8),
                                      nch)],
                    tbb.at[pl.ds(0, nch)])
            gissue(0, 0, use_c)

            def pair(j, _):
                k0 = 2 * j
                k1 = 2 * j + 1
                gwait(0)
                gissue(k1, 1, use_c)

                @pl.when(j > 0)
                def _():
                    swait(0)
                if use_c:
                    pltpu.sync_copy(rtab_hbm.at[tbb.at[k0]], bufC)
                compute(0, use_c)
                sissue(k0, 0)
                gwait(1)

                @pl.when(k1 + 1 < nch)
                def _():
                    gissue(k1 + 1, 0, use_c)

                @pl.when(j > 0)
                def _():
                    swait(1)
                if use_c:
                    pltpu.sync_copy(rtab_hbm.at[tbb.at[k1]], bufC)
                compute(1, use_c)
                sissue(k1, 1)
                return 0
            lax.fori_loop(0, nch // 2, pair, 0)
            swait(0)
            swait(1)
            return 0
        for hf in range(nhalves):
            half(hf, 0)

    run_phase(wid * C1W, 2, C1W // 2, False, 0)
    run_phase(E1P // K + wid * C2W, 1, C2W, True, wid * C2W)

    if with_mask:
        pltpu.sync_copy(mid_hbm.at[pl.ds(pl.multiple_of(wid * MPW, 8), MPW)],
                        imask)

        def mrow(r, _):
            for cc in range(8):
                obuf0[r, pl.ds(cc * 16, 16)] = z16
            obuf0[r, pl.ds(120, 16)] = z16
            return 0
        lax.fori_loop(0, MPW, mrow, 0)
        ones = jnp.ones((16,), jnp.float32)
        c130 = jnp.full((16,), 130, jnp.int32)
        for g in range(MPW // 16):
            rows = g * 16 + lax.iota(jnp.int32, 16)
            plsc.store_scatter(obuf0, [rows, c130], ones)
        pltpu.sync_copy(obuf0, acc.at[imask], add=True)

    plsc.subcore_barrier()

    def fsl(i, _):
        pltpu.sync_copy(acc.at[pl.ds(pl.multiple_of(zoff + i * K, 8), K)], obuf0)
        pltpu.sync_copy(
            obuf0,
            out_hbm.at[pl.ds(
                pl.multiple_of(c * ACC_ROWS + s * ZR + i * K, 8), K)])
        return 0
    lax.fori_loop(0, ZR // K, fsl, 0)


def _make_edge_kernel(nheads, with_mask):
    mesh = plsc.VectorSubcoreMesh(core_axis_name="c", subcore_axis_name="s",
                                  num_cores=NC, num_subcores=NS)
    return pl.kernel(
        functools.partial(_edge_body, nheads, with_mask),
        out_type=jax.ShapeDtypeStruct((NC * ACC_ROWS, TA), jnp.float32),
        mesh=mesh,
        scratch_types=[
            pltpu.VMEM_SHARED((ACC_ROWS, TA), jnp.float32),
            pltpu.VMEM((C1W // 2, K), jnp.int32),
            pltpu.VMEM((C1W // 2, K), jnp.int32),
            pltpu.VMEM((C1W // 2, K), jnp.int32),
            pltpu.VMEM((C2W, K), jnp.int32),
            pltpu.VMEM((MPW,), jnp.int32),
            pltpu.VMEM((K, TW), jnp.float32),
            pltpu.VMEM((K, TW), jnp.float32),
            pltpu.VMEM((K, TW), jnp.float32),
            pltpu.VMEM((K, TW), jnp.float32),
            pltpu.VMEM((K, 16), jnp.float32),
            pltpu.VMEM((K, 16), jnp.float32),
            pltpu.VMEM((K, TW), jnp.float32),
            pltpu.VMEM((K, TA), jnp.float32),
            pltpu.VMEM((K, TA), jnp.float32),
            pltpu.SemaphoreType.DMA,
            pltpu.SemaphoreType.DMA,
            pltpu.SemaphoreType.DMA,
            pltpu.SemaphoreType.DMA,
        ],
        compiler_params=pltpu.CompilerParams(use_tc_tiling_on_sc=False,
                                             needs_layout_passes=False),
    )


_BN = 1000
_BC = 80


def _stageA_body(x_ref, w_ref, y1_ref, y2_ref, y3_ref):
    x = x_ref[...]
    nrm = jnp.sqrt(jnp.sum(x * x, axis=1, keepdims=True))
    ent = x / jnp.maximum(nrm, 1e-12)
    z = jnp.dot(ent, w_ref[...], preferred_element_type=jnp.float32)
    y1_ref[...] = z[:, 0:256]
    y2_ref[:, 0:130] = z[:, 256:386]
    y2_ref[:, 130:TW] = jnp.zeros((_BN, TW - 130), jnp.float32)
    y3_ref[:, 0:2] = z[:, 386:388]
    y3_ref[:, 2:16] = jnp.zeros((_BN, 14), jnp.float32)


def _stageA(x, w):
    n = x.shape[0]
    return pl.pallas_call(
        _stageA_body,
        grid=(n // _BN,),
        in_specs=[pl.BlockSpec((_BN, x.shape[1]), lambda i: (i, 0)),
                  pl.BlockSpec(w.shape, lambda i: (0, 0))],
        out_specs=[pl.BlockSpec((_BN, 256), lambda i: (i, 0)),
                   pl.BlockSpec((_BN, TW), lambda i: (i, 0)),
                   pl.BlockSpec((_BN, 16), lambda i: (i, 0))],
        out_shape=[jax.ShapeDtypeStruct((n, 256), jnp.float32),
                   jax.ShapeDtypeStruct((n, TW), jnp.float32),
                   jax.ShapeDtypeStruct((n, 16), jnp.float32)],
    )(x, w)


def _stageR_body(x_ref, w_ref, y_ref):
    y_ref[...] = jnp.dot(x_ref[...], w_ref[...],
                         preferred_element_type=jnp.float32)


def _stageR(x, w):
    return pl.pallas_call(
        _stageR_body,
        out_shape=jax.ShapeDtypeStruct((x.shape[0], w.shape[1]), jnp.float32),
    )(x, w)


def _elu(v):
    return jnp.where(v > 0, v, jnp.exp(v) - 1.0)


def _stageC_body(pa_ref, pb_ref, p0_ref, w_ref, yq_ref, y2_ref, y3_ref):
    m = pa_ref[...] + pb_ref[...]
    rs = m[:, 128:130]
    rsr = jnp.where(rs == 0.0, 1e-12, rs)
    rse = jnp.concatenate([jnp.broadcast_to(rs[:, 0:1], (_BC, 64)),
                           jnp.broadcast_to(rs[:, 1:2], (_BC, 64))], axis=1)
    rsre = jnp.concatenate([jnp.broadcast_to(rsr[:, 0:1], (_BC, 64)),
                            jnp.broadcast_to(rsr[:, 1:2], (_BC, 64))], axis=1)
    x = _elu((rse * p0_ref[...] + m[:, :128]) / rsre)
    z = jnp.dot(x, w_ref[...], preferred_element_type=jnp.float32)
    yq_ref[...] = z[:, 0:128]
    y2_ref[:, 0:129] = z[:, 128:257]
    y2_ref[:, 129:TW] = jnp.zeros((_BC, TW - 129), jnp.float32)
    y3_ref[:, 0:1] = z[:, 257:258]
    y3_ref[:, 1:16] = jnp.zeros((_BC, 15), jnp.float32)


def _stageC(parts, p0eu, w):
    n = N_NODES
    return pl.pallas_call(
        _stageC_body,
        grid=(n // _BC,),
        in_specs=[pl.BlockSpec((_BC, TA), lambda i: (i, 0)),
                  pl.BlockSpec((_BC, TA), lambda i: (ACC_ROWS // _BC + i, 0)),
                  pl.BlockSpec((_BC, 128), lambda i: (i, 0)),
                  pl.BlockSpec(w.shape, lambda i: (0, 0))],
        out_specs=[pl.BlockSpec((_BC, 128), lambda i: (i, 0)),
                   pl.BlockSpec((_BC, TW), lambda i: (i, 0)),
                   pl.BlockSpec((_BC, 16), lambda i: (i, 0))],
        out_shape=[jax.ShapeDtypeStruct((n, 128), jnp.float32),
                   jax.ShapeDtypeStruct((n, TW), jnp.float32),
                   jax.ShapeDtypeStruct((n, 16), jnp.float32)],
    )(parts, parts, p0eu, w)


def _stageE_body(pa_ref, pb_ref, q0_ref, eu_ref, y_ref):
    m = pa_ref[...] + pb_ref[...]
    rs = m[:, 128:129]
    rsr = jnp.where(rs == 0.0, 1e-12, rs)
    x2 = _elu((rs * q0_ref[...] + m[:, :128]) / rsr)
    mask = (m[:, 130:131] > 0.0).astype(jnp.float32)
    o = eu_ref[...] + mask * x2
    nrm = jnp.sqrt(jnp.sum(o * o, axis=1, keepdims=True))
    y_ref[...] = o / jnp.maximum(nrm, 1e-12)


def _stageE(parts, q0, p0eu):
    n = N_NODES
    return pl.pallas_call(
        _stageE_body,
        grid=(n // _BC,),
        in_specs=[pl.BlockSpec((_BC, TA), lambda i: (i, 0)),
                  pl.BlockSpec((_BC, TA), lambda i: (ACC_ROWS // _BC + i, 0)),
                  pl.BlockSpec((_BC, 128), lambda i: (i, 0)),
                  pl.BlockSpec((_BC, 128), lambda i: (i, 1))],
        out_specs=pl.BlockSpec((_BC, 128), lambda i: (i, 0)),
        out_shape=jax.ShapeDtypeStruct((n, 128), jnp.float32),
    )(parts, parts, q0, p0eu)


def kernel(edge_list, edge_type, batch_inputs, train_indices_nhop,
           entity_embeddings, relation_embeddings, W_entities, W_rel,
           a_heads, a2_heads, a_out, a2_out, Corpus_=0, shuffle=0):
    f32 = jnp.float32
    uz = (jnp.asarray(Corpus_) + jnp.asarray(shuffle)).astype(f32)
    ent_in = entity_embeddings + uz

    nhop = train_indices_nhop
    p1 = E1P - E1
    p2 = E2P - E2
    src = jnp.concatenate([edge_list[0].astype(jnp.int32),
                           jnp.full((p1,), N_NODES, jnp.int32),
                           nhop[:, 3].astype(jnp.int32),
                           jnp.full((p2,), N_NODES, jnp.int32)])
    dst = jnp.concatenate([edge_list[1].astype(jnp.int32),
                           jnp.zeros((p1,), jnp.int32),
                           nhop[:, 0].astype(jnp.int32),
                           jnp.zeros((p2,), jnp.int32)])
    ta = jnp.concatenate([edge_type.astype(jnp.int32),
                          jnp.full((p1,), N_REL, jnp.int32),
                          nhop[:, 1].astype(jnp.int32),
                          jnp.full((p2,), N_REL, jnp.int32)])
    tb = jnp.concatenate([nhop[:, 2].astype(jnp.int32),
                          jnp.full((p2,), N_REL, jnp.int32)])
    src2d = src.reshape(-1, K)
    dst2d = dst.reshape(-1, K)
    ta2d = ta.reshape(-1, K)
    tb2d = tb.reshape(-1, K)
    mask_idx = batch_inputs[:MASK_B, 2].astype(jnp.int32)

    A0 = jnp.concatenate([a_heads[0][:, :128], a_heads[1][:, :128]], axis=0)
    A1 = jnp.concatenate([a_heads[0][:, 128:256], a_heads[1][:, 128:256]], axis=0)
    AR = jnp.concatenate([a_heads[0][:, 256:], a_heads[1][:, 256:]], axis=0)
    v0 = jnp.stack([a_heads[i][:, :128].T @ a2_heads[i][0] for i in range(2)], 1)
    v1 = jnp.stack([a_heads[i][:, 128:256].T @ a2_heads[i][0] for i in range(2)], 1)
    vr = jnp.stack([a_heads[i][:, 256:].T @ a2_heads[i][0] for i in range(2)], 1)
    B0 = a_out[:, :128]
    B1 = a_out[:, 128:256]
    BR = a_out[:, 256:]
    u0 = B0.T @ a2_out[0]
    u1 = B1.T @ a2_out[0]
    ur = BR.T @ a2_out[0]

    Wcat = jnp.concatenate([A0.T, W_entities, A1.T, v1, v0], axis=1)
    Wrcat = jnp.concatenate([AR.T, vr, W_rel, W_rel @ BR.T,
                             (W_rel @ ur)[:, None]], axis=1)

    p0eu, ptab1, stab1n = _stageA(ent_in, Wcat)
    Yr = _stageR(relation_embeddings, Wrcat)
    out_relation_1 = Yr[:, 130:258]

    zrel = jnp.zeros((1, TW), f32)
    zpad8 = jnp.zeros((8, 16), f32)

    rtab1 = jnp.concatenate(
        [jnp.concatenate([Yr[:, 0:128], Yr[:, 128:130],
                          jnp.zeros((N_REL, TW - 130), f32)], axis=1), zrel],
        axis=0)
    stab1 = jnp.concatenate([stab1n, zpad8], axis=0)

    part1 = _make_edge_kernel(2, False)(
        src2d, dst2d, ta2d, tb2d, mask_idx, ptab1, rtab1, stab1)

    Wc2 = jnp.concatenate([B0.T, B1.T, u1[:, None], u0[:, None]], axis=1)
    q0, ptab2, stab2n = _stageC(part1, p0eu, Wc2)

    rtab2 = jnp.concatenate(
        [jnp.concatenate([Yr[:, 258:386], Yr[:, 386:387],
                          jnp.zeros((N_REL, TW - 129), f32)], axis=1), zrel],
        axis=0)
    stab2 = jnp.concatenate([stab2n, zpad8], axis=0)

    part2 = _make_edge_kernel(1, True)(
        src2d, dst2d, ta2d, tb2d, mask_idx, ptab2, rtab2, stab2)

    out_entity_1 = _stageE(part2, q0, p0eu)
    return out_entity_1, out_relation_1
